# baseline (speedup 1.0000x reference)
import functools

import jax
import jax.numpy as jnp
from jax import lax
from jax.experimental import pallas as pl
from jax.experimental.pallas import tpu as pltpu
from jax.experimental.pallas import tpu_sc as plsc

_N = 64
_NQ = 262144
_NC = 2
_NS = 16
_NW = _NC * _NS
_L = 16
_CH = 1024
_QPW = _NQ // _NW
_NCHUNK = _QPW // _CH
_NGRP = _CH // _L

_ZP = 68
_PLANE = 66 * _ZP
_NLINE = 64 * 63
_LPW = _NLINE // _NW
_TROWS = _NLINE * 64
_LBLK = 21
_XSTRIDE = 63 * 64


def _mesh():
    return plsc.VectorSubcoreMesh(core_axis_name="c", subcore_axis_name="s")


_SC_PARAMS = pltpu.CompilerParams(
    needs_layout_passes=False, use_tc_tiling_on_sc=False)


def _build_table(fz_flat):
    @functools.partial(
        pl.kernel,
        mesh=_mesh(),
        compiler_params=_SC_PARAMS,
        out_type=jax.ShapeDtypeStruct((_TROWS, 16), jnp.int32),
        scratch_types=[
            pltpu.VMEM((4 * _PLANE,), jnp.float32),
            pltpu.VMEM((_LBLK * 64, 16), jnp.int32),
            pltpu.VMEM((_LBLK * 64, 16), jnp.int32),
            pltpu.SemaphoreType.DMA,
            pltpu.SemaphoreType.DMA,
        ],
    )
    def body(fz_hbm, tab_hbm, slab, stage0, stage1, semA, semB):
        wid = lax.axis_index("s") * _NC + lax.axis_index("c")
        lanes = lax.iota(jnp.int32, _L)
        line0 = wid * _LPW
        ix0 = 2 * wid
        pltpu.sync_copy(fz_hbm.at[pl.ds(ix0 * _PLANE, 4 * _PLANE)], slab)
        stages = (stage0, stage1)
        sems = (semA, semB)
        nblk = _LPW // _LBLK

        def build_block(blk, stage):
            @plsc.parallel_loop(0, _LBLK, unroll=2)
            def line_body(l):
                line = line0 + blk * _LBLK + l
                ix = line // 63
                jy = line - ix * 63
                for h in range(2):
                    poff = (ix - ix0 + 2 * h) * _PLANE + jy * _ZP
                    for b in range(4):
                        rowoff = poff + b * _ZP
                        for g in range(4):
                            ridx = (l * 64 + g * _L) + lanes
                            for cp in range(2):
                                va = slab[pl.ds(rowoff + g * _L + 2 * cp, _L)]
                                vb = slab[pl.ds(rowoff + g * _L + 2 * cp + 1, _L)]
                                w = plsc.bitcast(
                                    plsc.pack(va, vb, format=plsc.PackFormat.INTERLEAVED),
                                    jnp.int32)
                                cidx = jnp.full((_L,), h * 8 + b * 2 + cp, jnp.int32)
                                plsc.store_scatter(stage, [ridx, cidx], w)

        def out_slice(blk):
            return tab_hbm.at[pl.ds((line0 + blk * _LBLK) * 64, _LBLK * 64)]

        for blk in range(nblk):
            stage, sem = stages[blk % 2], sems[blk % 2]
            if blk >= 2:
                pltpu.make_async_copy(stage, out_slice(blk - 2), sem).wait()
            build_block(blk, stage)
            pltpu.async_copy(stage, out_slice(blk), sem)
        for blk in range(nblk - 2, nblk):
            pltpu.make_async_copy(stages[blk % 2], out_slice(blk), sems[blk % 2]).wait()

    return body(fz_flat)


def _tree_sum(terms):
    while len(terms) > 1:
        terms = [terms[i] + terms[i + 1] for i in range(0, len(terms) - 1, 2)] + (
            [terms[-1]] if len(terms) % 2 else [])
    return terms[0]


def _crw(t):
    t2 = t * t
    g = 0.5 * (t2 - t)
    w0 = g * (1.0 - t)
    w3 = g * t
    w1 = t2 * (1.5 * t - 2.5) + 1.0
    w2 = 1.0 - w0 - w1 - w3
    return w0, w1, w2, w3


def _sc_interp(table, xq, yq, zq):
    @functools.partial(
        pl.kernel,
        mesh=_mesh(),
        compiler_params=_SC_PARAMS,
        out_type=jax.ShapeDtypeStruct((_NQ,), jnp.float32),
        scratch_types=[
            pltpu.VMEM((_QPW,), jnp.float32),
            pltpu.VMEM((_QPW,), jnp.float32),
            pltpu.VMEM((_QPW,), jnp.float32),
            pltpu.VMEM((2 * _CH,), jnp.int32),
            pltpu.VMEM((2 * _CH,), jnp.int32),
            pltpu.VMEM((2 * _CH, 16), jnp.int32),
            pltpu.VMEM((2 * _CH, 16), jnp.int32),
            pltpu.VMEM((_QPW,), jnp.float32),
            pltpu.SemaphoreType.DMA,
            pltpu.SemaphoreType.DMA,
        ],
    )
    def body(p_hbm, xq_hbm, yq_hbm, zq_hbm, out_hbm,
             xv, yv, zv, idx0, idx1, rows0, rows1, ov, sem0, sem1):
        wid = lax.axis_index("s") * _NC + lax.axis_index("c")
        base = wid * _QPW
        pltpu.sync_copy(xq_hbm.at[pl.ds(base, _QPW)], xv)
        pltpu.sync_copy(yq_hbm.at[pl.ds(base, _QPW)], yv)
        pltpu.sync_copy(zq_hbm.at[pl.ds(base, _QPW)], zv)
        lanes = lax.iota(jnp.int32, _L)

        def idx_pass(ci, idxv):
            @plsc.parallel_loop(0, _NGRP, unroll=2)
            def g_body(g):
                off = ci * _CH + g * _L
                i = jnp.minimum(xv[pl.ds(off, _L)].astype(jnp.int32), _N - 2)
                j = jnp.minimum(yv[pl.ds(off, _L)].astype(jnp.int32), _N - 2)
                k = jnp.minimum(zv[pl.ds(off, _L)].astype(jnp.int32), _N - 2)
                r0 = (i * 63 + j) * 64 + k
                goff = g * _L
                idxv[pl.ds(goff, _L)] = r0
                idxv[pl.ds(_CH + goff, _L)] = r0 + _XSTRIDE


        def gather_start(idxv, rows, sem):
            pltpu.async_copy(p_hbm.at[idxv], rows, sem)

        def gather_wait(idxv, rows, sem):
            pltpu.make_async_copy(p_hbm.at[idxv], rows, sem).wait()

        def compute_pass(ci, rows):
            @plsc.parallel_loop(0, _NGRP, unroll=2)
            def g_body(g):
                off = ci * _CH + g * _L
                xs = xv[pl.ds(off, _L)]
                ys = yv[pl.ds(off, _L)]
                zs = zv[pl.ds(off, _L)]
                i = jnp.minimum(xs.astype(jnp.int32), _N - 2)
                j = jnp.minimum(ys.astype(jnp.int32), _N - 2)
                k = jnp.minimum(zs.astype(jnp.int32), _N - 2)
                wx = _crw(xs - i.astype(jnp.float32))
                wy = _crw(ys - j.astype(jnp.float32))
                wz = _crw(zs - k.astype(jnp.float32))
                wyz = [wy[b] * wz[c] for b in range(4) for c in range(4)]
                goff = g * _L
                acc_parts = []
                for p in range(2):
                    ridx = lanes + (p * _CH + goff)
                    for h in range(2):
                        a = p + 2 * h
                        terms = []
                        for b in range(4):
                            for cp in range(2):
                                cidx = jnp.full((_L,), h * 8 + b * 2 + cp, jnp.int32)
                                w = plsc.load_gather(rows, [ridx, cidx])
                                vlo, vhi = plsc.unpack(
                                    plsc.bitcast(w, jnp.bfloat16),
                                    format=plsc.PackFormat.INTERLEAVED)
                                bc = b * 4 + 2 * cp
                                terms.append(wyz[bc] * vlo.astype(jnp.float32))
                                terms.append(wyz[bc + 1] * vhi.astype(jnp.float32))
                        acc_parts.append(wx[a] * _tree_sum(terms))
                ov[pl.ds(off, _L)] = _tree_sum(acc_parts)


        idx_pass(0, idx0)
        gather_start(idx0, rows0, sem0)

        def pair_body(p, carry):
            c0 = 2 * p
            idx_pass(c0 + 1, idx1)
            gather_start(idx1, rows1, sem1)
            gather_wait(idx0, rows0, sem0)
            compute_pass(c0, rows0)

            @pl.when(c0 + 2 < _NCHUNK)
            def _():
                idx_pass(c0 + 2, idx0)
                gather_start(idx0, rows0, sem0)

            gather_wait(idx1, rows1, sem1)
            compute_pass(c0 + 1, rows1)
            return carry

        lax.fori_loop(0, _NCHUNK // 2, pair_body, 0)
        pltpu.sync_copy(ov, out_hbm.at[pl.ds(base, _QPW)])

    return body(table, xq, yq, zq)


def kernel(x, y, z, f, xq, yq, zq):
    def pad_axis(g, ax):
        g = jnp.moveaxis(g, ax, 0)
        g = jnp.concatenate([2.0 * g[0:1] - g[1:2], g, 2.0 * g[-1:] - g[-2:-1]], axis=0)
        return jnp.moveaxis(g, 0, ax)

    fp = pad_axis(pad_axis(pad_axis(f, 0), 1), 2)
    fz = jnp.pad(fp, ((0, 0), (0, 0), (0, _ZP - 66)))
    table = _build_table(fz.reshape(-1))
    return _sc_interp(table, xq.astype(jnp.float32), yq.astype(jnp.float32),
                      zq.astype(jnp.float32))


# speedup vs baseline: 1.0177x; 1.0177x over previous
import functools

import jax
import jax.numpy as jnp
from jax import lax
from jax.experimental import pallas as pl
from jax.experimental.pallas import tpu as pltpu
from jax.experimental.pallas import tpu_sc as plsc

_N = 64
_NQ = 262144
_NC = 2
_NS = 16
_NW = _NC * _NS
_L = 16
_CH = 512
_QPW = _NQ // _NW
_NCHUNK = _QPW // _CH
_NGRP = _CH // _L

_ZP = 68
_PLANE = 66 * _ZP
_NLINE = 64 * 63
_LPW = _NLINE // _NW
_TROWS = _NLINE * 64
_LBLK = 21
_XSTRIDE = 63 * 64


def _mesh():
    return plsc.VectorSubcoreMesh(core_axis_name="c", subcore_axis_name="s")


_SC_PARAMS = pltpu.CompilerParams(
    needs_layout_passes=False, use_tc_tiling_on_sc=False)


def _build_table(fz_flat):
    @functools.partial(
        pl.kernel,
        mesh=_mesh(),
        compiler_params=_SC_PARAMS,
        out_type=jax.ShapeDtypeStruct((_TROWS, 16), jnp.int32),
        scratch_types=[
            pltpu.VMEM((4 * _PLANE,), jnp.float32),
            pltpu.VMEM((_LBLK * 64, 16), jnp.int32),
            pltpu.VMEM((_LBLK * 64, 16), jnp.int32),
            pltpu.SemaphoreType.DMA,
            pltpu.SemaphoreType.DMA,
        ],
    )
    def body(fz_hbm, tab_hbm, slab, stage0, stage1, semA, semB):
        wid = lax.axis_index("s") * _NC + lax.axis_index("c")
        lanes = lax.iota(jnp.int32, _L)
        line0 = wid * _LPW
        ix0 = 2 * wid
        pltpu.sync_copy(fz_hbm.at[pl.ds(ix0 * _PLANE, 4 * _PLANE)], slab)
        stages = (stage0, stage1)
        sems = (semA, semB)
        nblk = _LPW // _LBLK

        def build_block(blk, stage):
            @plsc.parallel_loop(0, _LBLK, unroll=2)
            def line_body(l):
                line = line0 + blk * _LBLK + l
                ix = line // 63
                jy = line - ix * 63
                for h in range(2):
                    poff = (ix - ix0 + 2 * h) * _PLANE + jy * _ZP
                    for b in range(4):
                        rowoff = poff + b * _ZP
                        for g in range(4):
                            ridx = (l * 64 + g * _L) + lanes
                            for cp in range(2):
                                va = slab[pl.ds(rowoff + g * _L + 2 * cp, _L)]
                                vb = slab[pl.ds(rowoff + g * _L + 2 * cp + 1, _L)]
                                w = plsc.bitcast(
                                    plsc.pack(va, vb, format=plsc.PackFormat.INTERLEAVED),
                                    jnp.int32)
                                cidx = jnp.full((_L,), h * 8 + b * 2 + cp, jnp.int32)
                                plsc.store_scatter(stage, [ridx, cidx], w)

        def out_slice(blk):
            return tab_hbm.at[pl.ds((line0 + blk * _LBLK) * 64, _LBLK * 64)]

        for blk in range(nblk):
            stage, sem = stages[blk % 2], sems[blk % 2]
            if blk >= 2:
                pltpu.make_async_copy(stage, out_slice(blk - 2), sem).wait()
            build_block(blk, stage)
            pltpu.async_copy(stage, out_slice(blk), sem)
        for blk in range(nblk - 2, nblk):
            pltpu.make_async_copy(stages[blk % 2], out_slice(blk), sems[blk % 2]).wait()

    return body(fz_flat)


def _tree_sum(terms):
    while len(terms) > 1:
        terms = [terms[i] + terms[i + 1] for i in range(0, len(terms) - 1, 2)] + (
            [terms[-1]] if len(terms) % 2 else [])
    return terms[0]


def _crw(t):
    t2 = t * t
    g = 0.5 * (t2 - t)
    w0 = g * (1.0 - t)
    w3 = g * t
    w1 = t2 * (1.5 * t - 2.5) + 1.0
    w2 = 1.0 - w0 - w1 - w3
    return w0, w1, w2, w3


def _sc_interp(table, xq, yq, zq):
    @functools.partial(
        pl.kernel,
        mesh=_mesh(),
        compiler_params=_SC_PARAMS,
        out_type=jax.ShapeDtypeStruct((_NQ,), jnp.float32),
        scratch_types=[
            pltpu.VMEM((_QPW,), jnp.float32),
            pltpu.VMEM((_QPW,), jnp.float32),
            pltpu.VMEM((_QPW,), jnp.float32),
            pltpu.VMEM((2 * _CH,), jnp.int32),
            pltpu.VMEM((2 * _CH,), jnp.int32),
            pltpu.VMEM((2 * _CH, 16), jnp.int32),
            pltpu.VMEM((2 * _CH, 16), jnp.int32),
            pltpu.VMEM((_QPW,), jnp.float32),
            pltpu.SemaphoreType.DMA,
            pltpu.SemaphoreType.DMA,
            pltpu.SemaphoreType.DMA,
            pltpu.SemaphoreType.DMA,
        ],
    )
    def body(p_hbm, xq_hbm, yq_hbm, zq_hbm, out_hbm,
             xv, yv, zv, idx0, idx1, rows0, rows1, ov,
             sem0, sem1, sem0b, sem1b):
        wid = lax.axis_index("s") * _NC + lax.axis_index("c")
        base = wid * _QPW
        pltpu.sync_copy(xq_hbm.at[pl.ds(base, _QPW)], xv)
        pltpu.sync_copy(yq_hbm.at[pl.ds(base, _QPW)], yv)
        pltpu.sync_copy(zq_hbm.at[pl.ds(base, _QPW)], zv)
        lanes = lax.iota(jnp.int32, _L)

        def idx_pass(ci, idxv):
            @plsc.parallel_loop(0, _NGRP, unroll=2)
            def g_body(g):
                off = ci * _CH + g * _L
                i = jnp.minimum(xv[pl.ds(off, _L)].astype(jnp.int32), _N - 2)
                j = jnp.minimum(yv[pl.ds(off, _L)].astype(jnp.int32), _N - 2)
                k = jnp.minimum(zv[pl.ds(off, _L)].astype(jnp.int32), _N - 2)
                r0 = (i * 63 + j) * 64 + k
                goff = g * _L
                idxv[pl.ds(goff, _L)] = r0
                idxv[pl.ds(_CH + goff, _L)] = r0 + _XSTRIDE


        def gather_start(idxv, rows, sem, semb):
            pltpu.async_copy(p_hbm.at[idxv.at[pl.ds(0, _CH)]], rows.at[pl.ds(0, _CH)], sem)
            pltpu.async_copy(p_hbm.at[idxv.at[pl.ds(_CH, _CH)]], rows.at[pl.ds(_CH, _CH)], semb)

        def gather_wait(idxv, rows, sem, semb):
            pltpu.make_async_copy(p_hbm.at[idxv.at[pl.ds(0, _CH)]], rows.at[pl.ds(0, _CH)], sem).wait()
            pltpu.make_async_copy(p_hbm.at[idxv.at[pl.ds(_CH, _CH)]], rows.at[pl.ds(_CH, _CH)], semb).wait()

        def compute_pass(ci, rows):
            @plsc.parallel_loop(0, _NGRP, unroll=2)
            def g_body(g):
                off = ci * _CH + g * _L
                xs = xv[pl.ds(off, _L)]
                ys = yv[pl.ds(off, _L)]
                zs = zv[pl.ds(off, _L)]
                i = jnp.minimum(xs.astype(jnp.int32), _N - 2)
                j = jnp.minimum(ys.astype(jnp.int32), _N - 2)
                k = jnp.minimum(zs.astype(jnp.int32), _N - 2)
                wx = _crw(xs - i.astype(jnp.float32))
                wy = _crw(ys - j.astype(jnp.float32))
                wz = _crw(zs - k.astype(jnp.float32))
                wyz = [wy[b] * wz[c] for b in range(4) for c in range(4)]
                goff = g * _L
                acc_parts = []
                for p in range(2):
                    ridx = lanes + (p * _CH + goff)
                    for h in range(2):
                        a = p + 2 * h
                        terms = []
                        for b in range(4):
                            for cp in range(2):
                                cidx = jnp.full((_L,), h * 8 + b * 2 + cp, jnp.int32)
                                w = plsc.load_gather(rows, [ridx, cidx])
                                vlo, vhi = plsc.unpack(
                                    plsc.bitcast(w, jnp.bfloat16),
                                    format=plsc.PackFormat.INTERLEAVED)
                                bc = b * 4 + 2 * cp
                                terms.append(wyz[bc] * vlo.astype(jnp.float32))
                                terms.append(wyz[bc + 1] * vhi.astype(jnp.float32))
                        acc_parts.append(wx[a] * _tree_sum(terms))
                ov[pl.ds(off, _L)] = _tree_sum(acc_parts)


        idx_pass(0, idx0)
        gather_start(idx0, rows0, sem0, sem0b)

        def pair_body(p, carry):
            c0 = 2 * p
            idx_pass(c0 + 1, idx1)
            gather_start(idx1, rows1, sem1, sem1b)
            gather_wait(idx0, rows0, sem0, sem0b)
            compute_pass(c0, rows0)

            @pl.when(c0 + 2 < _NCHUNK)
            def _():
                idx_pass(c0 + 2, idx0)
                gather_start(idx0, rows0, sem0, sem0b)

            gather_wait(idx1, rows1, sem1, sem1b)
            compute_pass(c0 + 1, rows1)
            return carry

        lax.fori_loop(0, _NCHUNK // 2, pair_body, 0)
        pltpu.sync_copy(ov, out_hbm.at[pl.ds(base, _QPW)])

    return body(table, xq, yq, zq)


def kernel(x, y, z, f, xq, yq, zq):
    def pad_axis(g, ax):
        g = jnp.moveaxis(g, ax, 0)
        g = jnp.concatenate([2.0 * g[0:1] - g[1:2], g, 2.0 * g[-1:] - g[-2:-1]], axis=0)
        return jnp.moveaxis(g, 0, ax)

    fp = pad_axis(pad_axis(pad_axis(f, 0), 1), 2)
    fz = jnp.pad(fp, ((0, 0), (0, 0), (0, _ZP - 66)))
    table = _build_table(fz.reshape(-1))
    return _sc_interp(table, xq.astype(jnp.float32), yq.astype(jnp.float32),
                      zq.astype(jnp.float32))


# speedup vs baseline: 1.0255x; 1.0076x over previous
import functools

import jax
import jax.numpy as jnp
from jax import lax
from jax.experimental import pallas as pl
from jax.experimental.pallas import tpu as pltpu
from jax.experimental.pallas import tpu_sc as plsc

_N = 64
_NQ = 262144
_NC = 2
_NS = 16
_NW = _NC * _NS
_L = 16
_CH = 256
_QPW = _NQ // _NW
_NCHUNK = _QPW // _CH
_NGRP = _CH // _L

_ZP = 68
_PLANE = 66 * _ZP
_NLINE = 64 * 63
_LPW = _NLINE // _NW
_TROWS = _NLINE * 64
_LBLK = 21
_XSTRIDE = 63 * 64


def _mesh():
    return plsc.VectorSubcoreMesh(core_axis_name="c", subcore_axis_name="s")


_SC_PARAMS = pltpu.CompilerParams(
    needs_layout_passes=False, use_tc_tiling_on_sc=False)


def _build_table(fz_flat):
    @functools.partial(
        pl.kernel,
        mesh=_mesh(),
        compiler_params=_SC_PARAMS,
        out_type=jax.ShapeDtypeStruct((_TROWS, 16), jnp.int32),
        scratch_types=[
            pltpu.VMEM((4 * _PLANE,), jnp.float32),
            pltpu.VMEM((_LBLK * 64, 16), jnp.int32),
            pltpu.VMEM((_LBLK * 64, 16), jnp.int32),
            pltpu.SemaphoreType.DMA,
            pltpu.SemaphoreType.DMA,
        ],
    )
    def body(fz_hbm, tab_hbm, slab, stage0, stage1, semA, semB):
        wid = lax.axis_index("s") * _NC + lax.axis_index("c")
        lanes = lax.iota(jnp.int32, _L)
        line0 = wid * _LPW
        ix0 = 2 * wid
        pltpu.sync_copy(fz_hbm.at[pl.ds(ix0 * _PLANE, 4 * _PLANE)], slab)
        stages = (stage0, stage1)
        sems = (semA, semB)
        nblk = _LPW // _LBLK

        def build_block(blk, stage):
            @plsc.parallel_loop(0, _LBLK, unroll=2)
            def line_body(l):
                line = line0 + blk * _LBLK + l
                ix = line // 63
                jy = line - ix * 63
                for h in range(2):
                    poff = (ix - ix0 + 2 * h) * _PLANE + jy * _ZP
                    for b in range(4):
                        rowoff = poff + b * _ZP
                        for g in range(4):
                            ridx = (l * 64 + g * _L) + lanes
                            for cp in range(2):
                                va = slab[pl.ds(rowoff + g * _L + 2 * cp, _L)]
                                vb = slab[pl.ds(rowoff + g * _L + 2 * cp + 1, _L)]
                                w = plsc.bitcast(
                                    plsc.pack(va, vb, format=plsc.PackFormat.INTERLEAVED),
                                    jnp.int32)
                                cidx = jnp.full((_L,), h * 8 + b * 2 + cp, jnp.int32)
                                plsc.store_scatter(stage, [ridx, cidx], w)

        def out_slice(blk):
            return tab_hbm.at[pl.ds((line0 + blk * _LBLK) * 64, _LBLK * 64)]

        for blk in range(nblk):
            stage, sem = stages[blk % 2], sems[blk % 2]
            if blk >= 2:
                pltpu.make_async_copy(stage, out_slice(blk - 2), sem).wait()
            build_block(blk, stage)
            pltpu.async_copy(stage, out_slice(blk), sem)
        for blk in range(nblk - 2, nblk):
            pltpu.make_async_copy(stages[blk % 2], out_slice(blk), sems[blk % 2]).wait()

    return body(fz_flat)


def _tree_sum(terms):
    while len(terms) > 1:
        terms = [terms[i] + terms[i + 1] for i in range(0, len(terms) - 1, 2)] + (
            [terms[-1]] if len(terms) % 2 else [])
    return terms[0]


def _crw(t):
    t2 = t * t
    g = 0.5 * (t2 - t)
    w0 = g * (1.0 - t)
    w3 = g * t
    w1 = t2 * (1.5 * t - 2.5) + 1.0
    w2 = 1.0 - w0 - w1 - w3
    return w0, w1, w2, w3


def _sc_interp(table, xq, yq, zq):
    @functools.partial(
        pl.kernel,
        mesh=_mesh(),
        compiler_params=_SC_PARAMS,
        out_type=jax.ShapeDtypeStruct((_NQ,), jnp.float32),
        scratch_types=[
            pltpu.VMEM((_QPW,), jnp.float32),
            pltpu.VMEM((_QPW,), jnp.float32),
            pltpu.VMEM((_QPW,), jnp.float32),
            pltpu.VMEM((2 * _CH,), jnp.int32),
            pltpu.VMEM((2 * _CH,), jnp.int32),
            pltpu.VMEM((2 * _CH, 16), jnp.int32),
            pltpu.VMEM((2 * _CH, 16), jnp.int32),
            pltpu.VMEM((_QPW,), jnp.float32),
            pltpu.SemaphoreType.DMA,
            pltpu.SemaphoreType.DMA,
            pltpu.SemaphoreType.DMA,
            pltpu.SemaphoreType.DMA,
        ],
    )
    def body(p_hbm, xq_hbm, yq_hbm, zq_hbm, out_hbm,
             xv, yv, zv, idx0, idx1, rows0, rows1, ov,
             sem0, sem1, sem0b, sem1b):
        wid = lax.axis_index("s") * _NC + lax.axis_index("c")
        base = wid * _QPW
        pltpu.sync_copy(xq_hbm.at[pl.ds(base, _QPW)], xv)
        pltpu.sync_copy(yq_hbm.at[pl.ds(base, _QPW)], yv)
        pltpu.sync_copy(zq_hbm.at[pl.ds(base, _QPW)], zv)
        lanes = lax.iota(jnp.int32, _L)

        def idx_pass(ci, idxv):
            @plsc.parallel_loop(0, _NGRP, unroll=2)
            def g_body(g):
                off = ci * _CH + g * _L
                i = jnp.minimum(xv[pl.ds(off, _L)].astype(jnp.int32), _N - 2)
                j = jnp.minimum(yv[pl.ds(off, _L)].astype(jnp.int32), _N - 2)
                k = jnp.minimum(zv[pl.ds(off, _L)].astype(jnp.int32), _N - 2)
                r0 = (i * 63 + j) * 64 + k
                goff = g * _L
                idxv[pl.ds(goff, _L)] = r0
                idxv[pl.ds(_CH + goff, _L)] = r0 + _XSTRIDE


        def gather_start(idxv, rows, sem, semb):
            pltpu.async_copy(p_hbm.at[idxv.at[pl.ds(0, _CH)]], rows.at[pl.ds(0, _CH)], sem)
            pltpu.async_copy(p_hbm.at[idxv.at[pl.ds(_CH, _CH)]], rows.at[pl.ds(_CH, _CH)], semb)

        def gather_wait(idxv, rows, sem, semb):
            pltpu.make_async_copy(p_hbm.at[idxv.at[pl.ds(0, _CH)]], rows.at[pl.ds(0, _CH)], sem).wait()
            pltpu.make_async_copy(p_hbm.at[idxv.at[pl.ds(_CH, _CH)]], rows.at[pl.ds(_CH, _CH)], semb).wait()

        def compute_pass(ci, rows):
            @plsc.parallel_loop(0, _NGRP, unroll=2)
            def g_body(g):
                off = ci * _CH + g * _L
                xs = xv[pl.ds(off, _L)]
                ys = yv[pl.ds(off, _L)]
                zs = zv[pl.ds(off, _L)]
                i = jnp.minimum(xs.astype(jnp.int32), _N - 2)
                j = jnp.minimum(ys.astype(jnp.int32), _N - 2)
                k = jnp.minimum(zs.astype(jnp.int32), _N - 2)
                wx = _crw(xs - i.astype(jnp.float32))
                wy = _crw(ys - j.astype(jnp.float32))
                wz = _crw(zs - k.astype(jnp.float32))
                wyz = [wy[b] * wz[c] for b in range(4) for c in range(4)]
                goff = g * _L
                acc_parts = []
                for p in range(2):
                    ridx = lanes + (p * _CH + goff)
                    for h in range(2):
                        a = p + 2 * h
                        terms = []
                        for b in range(4):
                            for cp in range(2):
                                cidx = jnp.full((_L,), h * 8 + b * 2 + cp, jnp.int32)
                                w = plsc.load_gather(rows, [ridx, cidx])
                                vlo, vhi = plsc.unpack(
                                    plsc.bitcast(w, jnp.bfloat16),
                                    format=plsc.PackFormat.INTERLEAVED)
                                bc = b * 4 + 2 * cp
                                terms.append(wyz[bc] * vlo.astype(jnp.float32))
                                terms.append(wyz[bc + 1] * vhi.astype(jnp.float32))
                        acc_parts.append(wx[a] * _tree_sum(terms))
                ov[pl.ds(off, _L)] = _tree_sum(acc_parts)


        idx_pass(0, idx0)
        gather_start(idx0, rows0, sem0, sem0b)

        def pair_body(p, carry):
            c0 = 2 * p
            idx_pass(c0 + 1, idx1)
            gather_start(idx1, rows1, sem1, sem1b)
            gather_wait(idx0, rows0, sem0, sem0b)
            compute_pass(c0, rows0)

            @pl.when(c0 + 2 < _NCHUNK)
            def _():
                idx_pass(c0 + 2, idx0)
                gather_start(idx0, rows0, sem0, sem0b)

            gather_wait(idx1, rows1, sem1, sem1b)
            compute_pass(c0 + 1, rows1)
            return carry

        lax.fori_loop(0, _NCHUNK // 2, pair_body, 0)
        pltpu.sync_copy(ov, out_hbm.at[pl.ds(base, _QPW)])

    return body(table, xq, yq, zq)


def kernel(x, y, z, f, xq, yq, zq):
    def pad_axis(g, ax):
        g = jnp.moveaxis(g, ax, 0)
        g = jnp.concatenate([2.0 * g[0:1] - g[1:2], g, 2.0 * g[-1:] - g[-2:-1]], axis=0)
        return jnp.moveaxis(g, 0, ax)

    fp = pad_axis(pad_axis(pad_axis(f, 0), 1), 2)
    fz = jnp.pad(fp, ((0, 0), (0, 0), (0, _ZP - 66)))
    table = _build_table(fz.reshape(-1))
    return _sc_interp(table, xq.astype(jnp.float32), yq.astype(jnp.float32),
                      zq.astype(jnp.float32))


# speedup vs baseline: 1.0307x; 1.0050x over previous
import functools

import jax
import jax.numpy as jnp
from jax import lax
from jax.experimental import pallas as pl
from jax.experimental.pallas import tpu as pltpu
from jax.experimental.pallas import tpu_sc as plsc

_N = 64
_NQ = 262144
_NC = 2
_NS = 16
_NW = _NC * _NS
_L = 16
_CH = 128
_QPW = _NQ // _NW
_NCHUNK = _QPW // _CH
_NGRP = _CH // _L

_ZP = 68
_PLANE = 66 * _ZP
_NLINE = 64 * 63
_LPW = _NLINE // _NW
_TROWS = _NLINE * 64
_LBLK = 21
_XSTRIDE = 63 * 64


def _mesh():
    return plsc.VectorSubcoreMesh(core_axis_name="c", subcore_axis_name="s")


_SC_PARAMS = pltpu.CompilerParams(
    needs_layout_passes=False, use_tc_tiling_on_sc=False)


def _build_table(fz_flat):
    @functools.partial(
        pl.kernel,
        mesh=_mesh(),
        compiler_params=_SC_PARAMS,
        out_type=jax.ShapeDtypeStruct((_TROWS, 16), jnp.int32),
        scratch_types=[
            pltpu.VMEM((4 * _PLANE,), jnp.float32),
            pltpu.VMEM((_LBLK * 64, 16), jnp.int32),
            pltpu.VMEM((_LBLK * 64, 16), jnp.int32),
            pltpu.SemaphoreType.DMA,
            pltpu.SemaphoreType.DMA,
        ],
    )
    def body(fz_hbm, tab_hbm, slab, stage0, stage1, semA, semB):
        wid = lax.axis_index("s") * _NC + lax.axis_index("c")
        lanes = lax.iota(jnp.int32, _L)
        line0 = wid * _LPW
        ix0 = 2 * wid
        pltpu.sync_copy(fz_hbm.at[pl.ds(ix0 * _PLANE, 4 * _PLANE)], slab)
        stages = (stage0, stage1)
        sems = (semA, semB)
        nblk = _LPW // _LBLK

        def build_block(blk, stage):
            @plsc.parallel_loop(0, _LBLK, unroll=2)
            def line_body(l):
                line = line0 + blk * _LBLK + l
                ix = line // 63
                jy = line - ix * 63
                for h in range(2):
                    poff = (ix - ix0 + 2 * h) * _PLANE + jy * _ZP
                    for b in range(4):
                        rowoff = poff + b * _ZP
                        for g in range(4):
                            ridx = (l * 64 + g * _L) + lanes
                            for cp in range(2):
                                va = slab[pl.ds(rowoff + g * _L + 2 * cp, _L)]
                                vb = slab[pl.ds(rowoff + g * _L + 2 * cp + 1, _L)]
                                w = plsc.bitcast(
                                    plsc.pack(va, vb, format=plsc.PackFormat.INTERLEAVED),
                                    jnp.int32)
                                cidx = jnp.full((_L,), h * 8 + b * 2 + cp, jnp.int32)
                                plsc.store_scatter(stage, [ridx, cidx], w)

        def out_slice(blk):
            return tab_hbm.at[pl.ds((line0 + blk * _LBLK) * 64, _LBLK * 64)]

        for blk in range(nblk):
            stage, sem = stages[blk % 2], sems[blk % 2]
            if blk >= 2:
                pltpu.make_async_copy(stage, out_slice(blk - 2), sem).wait()
            build_block(blk, stage)
            pltpu.async_copy(stage, out_slice(blk), sem)
        for blk in range(nblk - 2, nblk):
            pltpu.make_async_copy(stages[blk % 2], out_slice(blk), sems[blk % 2]).wait()

    return body(fz_flat)


def _tree_sum(terms):
    while len(terms) > 1:
        terms = [terms[i] + terms[i + 1] for i in range(0, len(terms) - 1, 2)] + (
            [terms[-1]] if len(terms) % 2 else [])
    return terms[0]


def _crw(t):
    t2 = t * t
    g = 0.5 * (t2 - t)
    w0 = g * (1.0 - t)
    w3 = g * t
    w1 = t2 * (1.5 * t - 2.5) + 1.0
    w2 = 1.0 - w0 - w1 - w3
    return w0, w1, w2, w3


def _sc_interp(table, xq, yq, zq):
    @functools.partial(
        pl.kernel,
        mesh=_mesh(),
        compiler_params=_SC_PARAMS,
        out_type=jax.ShapeDtypeStruct((_NQ,), jnp.float32),
        scratch_types=[
            pltpu.VMEM((_QPW,), jnp.float32),
            pltpu.VMEM((_QPW,), jnp.float32),
            pltpu.VMEM((_QPW,), jnp.float32),
            pltpu.VMEM((2 * _CH,), jnp.int32),
            pltpu.VMEM((2 * _CH,), jnp.int32),
            pltpu.VMEM((2 * _CH, 16), jnp.int32),
            pltpu.VMEM((2 * _CH, 16), jnp.int32),
            pltpu.VMEM((_QPW,), jnp.float32),
            pltpu.SemaphoreType.DMA,
            pltpu.SemaphoreType.DMA,
            pltpu.SemaphoreType.DMA,
            pltpu.SemaphoreType.DMA,
        ],
    )
    def body(p_hbm, xq_hbm, yq_hbm, zq_hbm, out_hbm,
             xv, yv, zv, idx0, idx1, rows0, rows1, ov,
             sem0, sem1, sem0b, sem1b):
        wid = lax.axis_index("s") * _NC + lax.axis_index("c")
        base = wid * _QPW
        pltpu.sync_copy(xq_hbm.at[pl.ds(base, _QPW)], xv)
        pltpu.sync_copy(yq_hbm.at[pl.ds(base, _QPW)], yv)
        pltpu.sync_copy(zq_hbm.at[pl.ds(base, _QPW)], zv)
        lanes = lax.iota(jnp.int32, _L)

        def idx_pass(ci, idxv):
            @plsc.parallel_loop(0, _NGRP, unroll=2)
            def g_body(g):
                off = ci * _CH + g * _L
                i = jnp.minimum(xv[pl.ds(off, _L)].astype(jnp.int32), _N - 2)
                j = jnp.minimum(yv[pl.ds(off, _L)].astype(jnp.int32), _N - 2)
                k = jnp.minimum(zv[pl.ds(off, _L)].astype(jnp.int32), _N - 2)
                r0 = (i * 63 + j) * 64 + k
                goff = g * _L
                idxv[pl.ds(goff, _L)] = r0
                idxv[pl.ds(_CH + goff, _L)] = r0 + _XSTRIDE


        def gather_start(idxv, rows, sem, semb):
            pltpu.async_copy(p_hbm.at[idxv.at[pl.ds(0, _CH)]], rows.at[pl.ds(0, _CH)], sem)
            pltpu.async_copy(p_hbm.at[idxv.at[pl.ds(_CH, _CH)]], rows.at[pl.ds(_CH, _CH)], semb)

        def gather_wait(idxv, rows, sem, semb):
            pltpu.make_async_copy(p_hbm.at[idxv.at[pl.ds(0, _CH)]], rows.at[pl.ds(0, _CH)], sem).wait()
            pltpu.make_async_copy(p_hbm.at[idxv.at[pl.ds(_CH, _CH)]], rows.at[pl.ds(_CH, _CH)], semb).wait()

        def compute_pass(ci, rows):
            @plsc.parallel_loop(0, _NGRP, unroll=2)
            def g_body(g):
                off = ci * _CH + g * _L
                xs = xv[pl.ds(off, _L)]
                ys = yv[pl.ds(off, _L)]
                zs = zv[pl.ds(off, _L)]
                i = jnp.minimum(xs.astype(jnp.int32), _N - 2)
                j = jnp.minimum(ys.astype(jnp.int32), _N - 2)
                k = jnp.minimum(zs.astype(jnp.int32), _N - 2)
                wx = _crw(xs - i.astype(jnp.float32))
                wy = _crw(ys - j.astype(jnp.float32))
                wz = _crw(zs - k.astype(jnp.float32))
                wyz = [wy[b] * wz[c] for b in range(4) for c in range(4)]
                goff = g * _L
                acc_parts = []
                for p in range(2):
                    ridx = lanes + (p * _CH + goff)
                    for h in range(2):
                        a = p + 2 * h
                        terms = []
                        for b in range(4):
                            for cp in range(2):
                                cidx = jnp.full((_L,), h * 8 + b * 2 + cp, jnp.int32)
                                w = plsc.load_gather(rows, [ridx, cidx])
                                vlo, vhi = plsc.unpack(
                                    plsc.bitcast(w, jnp.bfloat16),
                                    format=plsc.PackFormat.INTERLEAVED)
                                bc = b * 4 + 2 * cp
                                terms.append(wyz[bc] * vlo.astype(jnp.float32))
                                terms.append(wyz[bc + 1] * vhi.astype(jnp.float32))
                        acc_parts.append(wx[a] * _tree_sum(terms))
                ov[pl.ds(off, _L)] = _tree_sum(acc_parts)


        idx_pass(0, idx0)
        gather_start(idx0, rows0, sem0, sem0b)

        def pair_body(p, carry):
            c0 = 2 * p
            idx_pass(c0 + 1, idx1)
            gather_start(idx1, rows1, sem1, sem1b)
            gather_wait(idx0, rows0, sem0, sem0b)
            compute_pass(c0, rows0)

            @pl.when(c0 + 2 < _NCHUNK)
            def _():
                idx_pass(c0 + 2, idx0)
                gather_start(idx0, rows0, sem0, sem0b)

            gather_wait(idx1, rows1, sem1, sem1b)
            compute_pass(c0 + 1, rows1)
            return carry

        lax.fori_loop(0, _NCHUNK // 2, pair_body, 0)
        pltpu.sync_copy(ov, out_hbm.at[pl.ds(base, _QPW)])

    return body(table, xq, yq, zq)


def kernel(x, y, z, f, xq, yq, zq):
    def pad_axis(g, ax):
        g = jnp.moveaxis(g, ax, 0)
        g = jnp.concatenate([2.0 * g[0:1] - g[1:2], g, 2.0 * g[-1:] - g[-2:-1]], axis=0)
        return jnp.moveaxis(g, 0, ax)

    fp = pad_axis(pad_axis(pad_axis(f, 0), 1), 2)
    fz = jnp.pad(fp, ((0, 0), (0, 0), (0, _ZP - 66)))
    table = _build_table(fz.reshape(-1))
    return _sc_interp(table, xq.astype(jnp.float32), yq.astype(jnp.float32),
                      zq.astype(jnp.float32))
